# baseline (speedup 1.0000x reference)
import functools

import jax
import jax.numpy as jnp
from jax import lax
from jax.experimental import pallas as pl
from jax.experimental.pallas import tpu as pltpu
from jax.experimental.pallas import tpu_sc as plsc

B = 2048
IN = 256
H = 8192
OUT = 256
HBLK = 512
NBLK = H // HBLK


def _tc_body(x_ref, w_ref, g_ref, win_ref, gt_ref, minv, argv):
    j = pl.program_id(0)
    x = x_ref[...]
    w = w_ref[...]
    s = lax.dot_general(x, w, (((1,), (1,)), ((), ())),
                        preferred_element_type=jnp.float32)
    x2 = jnp.sum(x * x, axis=1, keepdims=True)
    w2 = jnp.sum(w * w, axis=1)
    d2 = jnp.maximum(x2 + w2[None, :] - 2.0 * s, 0.0)
    bmin = jnp.min(d2, axis=1, keepdims=True)
    barg = (jnp.argmin(d2, axis=1).astype(jnp.int32)[:, None]
            + j * HBLK)

    @pl.when(j == 0)
    def _():
        minv[...] = bmin
        argv[...] = barg

    @pl.when(j > 0)
    def _():
        upd = bmin < minv[...]
        minv[...] = jnp.where(upd, bmin, minv[...])
        argv[...] = jnp.where(upd, barg, argv[...])

    @pl.when(j == NBLK - 1)
    def _():
        win_ref[...] = argv[...]

    gt_ref[...] = g_ref[...].T


def _tc_call(x, kw, gw):
    return pl.pallas_call(
        _tc_body,
        grid=(NBLK,),
        in_specs=[
            pl.BlockSpec((B, IN), lambda j: (0, 0)),
            pl.BlockSpec((HBLK, IN), lambda j: (j, 0)),
            pl.BlockSpec((OUT, HBLK), lambda j: (0, j)),
        ],
        out_specs=[
            pl.BlockSpec((B, 1), lambda j: (0, 0)),
            pl.BlockSpec((HBLK, OUT), lambda j: (j, 0)),
        ],
        out_shape=[
            jax.ShapeDtypeStruct((B, 1), jnp.int32),
            jax.ShapeDtypeStruct((H, OUT), jnp.float32),
        ],
        scratch_shapes=[
            pltpu.VMEM((B, 1), jnp.float32),
            pltpu.VMEM((B, 1), jnp.int32),
        ],
    )(x, kw, gw)


_NC = 2
_NS = 16
_NW = _NC * _NS
_BPW = B // _NW


@functools.cache
def _make_sc_gather():
    @functools.partial(
        pl.kernel,
        mesh=plsc.VectorSubcoreMesh(core_axis_name="c", subcore_axis_name="s"),
        out_type=jax.ShapeDtypeStruct((B, OUT), jnp.float32),
        scratch_types=[
            pltpu.VMEM((_BPW,), jnp.int32),
            pltpu.VMEM((_BPW, OUT), jnp.float32),
            pltpu.SemaphoreType.DMA,
        ],
    )
    def _sc_gather(table_hbm, idx_hbm, out_hbm, idx_v, rows_v, sem):
        wid = lax.axis_index("s") * _NC + lax.axis_index("c")
        base = wid * _BPW
        pltpu.sync_copy(idx_hbm.at[pl.ds(base, _BPW)], idx_v)
        pltpu.async_copy(table_hbm.at[idx_v], rows_v, sem).wait()
        pltpu.sync_copy(rows_v, out_hbm.at[pl.ds(base, _BPW)])

    return _sc_gather


def kernel(x, kohonen_weights, grossberg_weights):
    win2d, gt = _tc_call(x, kohonen_weights, grossberg_weights)
    winners = win2d.reshape(B)
    output = _make_sc_gather()(gt, winners)
    return (output, winners)
